# baseline (speedup 1.0000x reference)
import functools

import jax
import jax.numpy as jnp
from jax import lax
from jax.experimental import pallas as pl
from jax.experimental.pallas import tpu as pltpu
from jax.experimental.pallas import tpu_sc as plsc

N_NODES = 100000
N_EDGES = 3200000
EDGE_IN = 4
HIDDEN = 128
NODE_OUT = 128
N_GRAPHS = 16
BN = 1000
GRID = N_NODES // BN

SC_NC = 2
SC_NS = 16
SC_NW = SC_NC * SC_NS
E_PER_W = N_EDGES // SC_NW
ROWW = 80
CHUNK = 800
ROWS_PER_CHUNK = CHUNK // ROWW
NCHUNK = E_PER_W // CHUNK
NEG = -1e30
HALFN = N_NODES // 2
NGROUP = 8
E_PER_G = N_EDGES // SC_NC // NGROUP
CHUNK_M = 4000
NCHUNK_M = E_PER_G // CHUNK_M
NP_MAX = SC_NC * NGROUP
AW = 128


def _sc_body(col2d_h, attrw_h, z8_h,
             sump_h, maxp_h,
             idx2_v, attrw_v, pay8_v, idxf_v, table_v,
             t8_sh, sem):
    c = lax.axis_index("c")
    s = lax.axis_index("s")
    w = c * SC_NS + s
    lanes = lax.iota(jnp.int32, 16)

    @pl.when(s == 0)
    def _():
        pltpu.sync_copy(z8_h, t8_sh)
    plsc.subcore_barrier()

    e16 = lanes
    def _init_pay(i, carry):
        ei = i * 16 + e16
        plsc.store_scatter(pay8_v, [ei, jnp.full((16,), 4, jnp.int32)],
                           jnp.ones((16,), jnp.float32))
        for q in (5, 6, 7):
            plsc.store_scatter(pay8_v, [ei, jnp.full((16,), q, jnp.int32)],
                               jnp.zeros((16,), jnp.float32))
        return carry
    lax.fori_loop(0, CHUNK // 16, _init_pay, 0)

    def _sum_chunk(ci, carry):
        rowbase = (w * E_PER_W + ci * CHUNK) // ROWW
        wordrow = (w * E_PER_W + ci * CHUNK) * 4 // AW
        pltpu.sync_copy(col2d_h.at[pl.ds(rowbase, ROWS_PER_CHUNK), :], idx2_v)
        pltpu.sync_copy(attrw_h.at[pl.ds(wordrow, CHUNK * 4 // AW), :],
                        attrw_v.at[pl.ds(0, CHUNK * 4 // AW), :])

        def _fill(k, carry2):
            t = k * 16 + lanes
            a16 = attrw_v[k // (AW // 16), pl.ds((k % (AW // 16)) * 16, 16)]
            plsc.store_scatter(pay8_v, [t >> 2, t & 3], a16)
            return carry2
        lax.fori_loop(0, CHUNK * 4 // 16, _fill, 0)

        handles = []
        for j in range(ROWS_PER_CHUNK):
            handles.append(pltpu.async_copy(
                pay8_v.at[pl.ds(j * ROWW, ROWW), :],
                t8_sh.at[idx2_v.at[j]], sem, add=True))
        for h in handles:
            h.wait()
        return carry
    lax.fori_loop(0, NCHUNK, _sum_chunk, 0)
    plsc.subcore_barrier()

    def _sout(k, carry):
        gb = s + k * SC_NS
        @pl.when(gb < GRID)
        def _():
            pltpu.sync_copy(t8_sh.at[pl.ds(gb * BN, BN), :],
                            sump_h.at[gb, c, :, :])
        return carry
    lax.fori_loop(0, (GRID + SC_NS - 1) // SC_NS, _sout, 0)

    g = s // 2
    r = s % 2
    lo = r * HALFN
    neg16 = jnp.full((16,), NEG, jnp.float32)
    for f in range(EDGE_IN):
        def _init_tab(i, carry):
            table_v[pl.ds(i * 16, 16)] = neg16
            return carry
        lax.fori_loop(0, HALFN // 16, _init_tab, 0)

        lanes4f = lanes * 4 + f

        def _max_chunk(ci, carry):
            ebase = c * (N_EDGES // SC_NC) + g * E_PER_G + ci * CHUNK_M
            pltpu.sync_copy(
                col2d_h.at[pl.ds(ebase // ROWW, CHUNK_M // ROWW), :], idxf_v)
            pltpu.sync_copy(
                attrw_h.at[pl.ds(ebase * 4 // AW, CHUNK_M * 4 // AW), :],
                attrw_v)

            def _vec(jv, vcarry):
                i16 = idxf_v[jv // (ROWW // 16), pl.ds((jv % (ROWW // 16)) * 16, 16)]
                t = jv * 64 + lanes4f
                v16 = plsc.load_gather(attrw_v, [t >> 7, t & (AW - 1)])
                il = i16 - lo
                inr = il.astype(jnp.uint32) < jnp.uint32(HALFN)
                old = plsc.load_gather(table_v, [il], mask=inr)
                plsc.store_scatter(table_v, [il], jnp.maximum(old, v16),
                                   mask=inr)
                m0 = inr & (plsc.load_gather(table_v, [il], mask=inr) < v16)

                def _cond(m):
                    return jnp.any(m)

                def _body(m):
                    cur = plsc.load_gather(table_v, [il], mask=m)
                    plsc.store_scatter(table_v, [il],
                                       jnp.maximum(cur, v16), mask=m)
                    return m & (plsc.load_gather(table_v, [il], mask=m)
                                < v16)
                lax.while_loop(_cond, _body, m0)
                return vcarry
            lax.fori_loop(0, CHUNK_M // 16, _vec, 0)
            return carry
        lax.fori_loop(0, NCHUNK_M, _max_chunk, 0)

        p = c * NGROUP + g
        def _wout(k, carry):
            pltpu.sync_copy(table_v.at[pl.ds(k * BN, BN)],
                            maxp_h.at[r * (HALFN // BN) + k, p, f, :])
            return carry
        lax.fori_loop(0, HALFN // BN, _wout, 0)


def _sc_scatter(col2d, attrw, z8):
    mesh = plsc.VectorSubcoreMesh(core_axis_name="c", subcore_axis_name="s")
    return pl.kernel(
        _sc_body,
        out_type=[
            jax.ShapeDtypeStruct((GRID, SC_NC, BN, 8), jnp.float32),
            jax.ShapeDtypeStruct((GRID, NP_MAX, 4, BN), jnp.float32),
        ],
        mesh=mesh,
        compiler_params=pltpu.CompilerParams(use_tc_tiling_on_sc=False,
                                             needs_layout_passes=False),
        scratch_types=(
            [pltpu.VMEM((ROWS_PER_CHUNK, ROWW), jnp.int32),
             pltpu.VMEM((CHUNK_M * 4 // AW, AW), jnp.float32),
             pltpu.VMEM((CHUNK, 8), jnp.float32),
             pltpu.VMEM((CHUNK_M // ROWW, ROWW), jnp.int32),
             pltpu.VMEM((HALFN,), jnp.float32),
             pltpu.VMEM_SHARED((N_NODES, 8), jnp.float32),
             pltpu.SemaphoreType.DMA]
        ),
    )(col2d, attrw, z8)


def _mlp_body(x_ref, sump_ref, maxp_ref, b2d_ref, u_ref,
              w1a_ref, w1b1_ref, w1b2_ref, w1b3_ref, w1c_ref, b1_ref,
              w2_ref, bias2_ref, out_ref):
    f32 = jnp.float32
    sall = jnp.sum(sump_ref[...], axis=1).reshape(BN, 8)
    out1 = sall[:, :4]
    cnt2 = sall[:, 4:5]
    mx = jnp.max(maxp_ref[...], axis=1).reshape(4, BN)
    out2t = jnp.where(mx > (0.5 * NEG), mx, 0.0)
    out3 = out1 * (1.0 / jnp.maximum(cnt2, 1.0))

    acc = jnp.dot(x_ref[...], w1a_ref[...], preferred_element_type=f32)
    acc += b1_ref[...]
    acc += jnp.dot(out1, w1b1_ref[...], preferred_element_type=f32)
    acc += lax.dot_general(out2t, w1b2_ref[...],
                           dimension_numbers=(((0,), (0,)), ((), ())),
                           preferred_element_type=f32)
    acc += jnp.dot(out3, w1b3_ref[...], preferred_element_type=f32)
    oh = (lax.broadcasted_iota(jnp.int32, (BN, N_GRAPHS), 1)
          == b2d_ref[...]).astype(f32)
    uw = jnp.dot(u_ref[...], w1c_ref[...], preferred_element_type=f32)
    acc += jnp.dot(oh, uw, preferred_element_type=f32)
    h1 = jnp.maximum(acc, 0.0)
    out_ref[...] = (jnp.dot(h1, w2_ref[...], preferred_element_type=f32)
                    + bias2_ref[...])


def _mlp_call(x, sump, maxp, b2d, u, W1, b1, W2, b2):
    p1 = sump.shape[1]
    p2 = maxp.shape[1]
    full = lambda shape: pl.BlockSpec(shape, lambda i: tuple(0 for _ in shape))
    return pl.pallas_call(
        _mlp_body,
        grid=(GRID,),
        in_specs=[
            pl.BlockSpec((BN, 128), lambda i: (i, 0)),
            pl.BlockSpec((1, p1, BN, 8), lambda i: (i, 0, 0, 0)),
            pl.BlockSpec((1, p2, 4, BN), lambda i: (i, 0, 0, 0)),
            pl.BlockSpec((BN, 1), lambda i: (i, 0)),
            full((N_GRAPHS, 16)),
            full((128, HIDDEN)),
            full((4, HIDDEN)),
            full((4, HIDDEN)),
            full((4, HIDDEN)),
            full((16, HIDDEN)),
            full((1, HIDDEN)),
            full((HIDDEN, NODE_OUT)),
            full((1, NODE_OUT)),
        ],
        out_specs=pl.BlockSpec((BN, NODE_OUT), lambda i: (i, 0)),
        out_shape=jax.ShapeDtypeStruct((N_NODES, NODE_OUT), jnp.float32),
        compiler_params=pltpu.CompilerParams(
            dimension_semantics=("arbitrary",)),
    )(x, sump, maxp, b2d, u, W1[:128], W1[128:132], W1[132:136],
      W1[136:140], W1[140:156], b1.reshape(1, HIDDEN), W2,
      b2.reshape(1, NODE_OUT))


def kernel(x, edge_index, edge_attr, u, batch, W1, b1, W2, b2):
    col2d = edge_index[1].astype(jnp.int32).reshape(N_EDGES // ROWW, ROWW)
    z8 = jnp.zeros((N_NODES, 8), jnp.float32)
    attrw = edge_attr.reshape(N_EDGES * 4 // AW, AW)
    sump, maxp = _sc_scatter(col2d, attrw, z8)
    b2d = batch.astype(jnp.int32).reshape(N_NODES, 1)
    return _mlp_call(x, sump, maxp, b2d, u, W1, b1, W2, b2)


# speedup vs baseline: 2.5720x; 2.5720x over previous
import functools

import jax
import jax.numpy as jnp
from jax import lax
from jax.experimental import pallas as pl
from jax.experimental.pallas import tpu as pltpu
from jax.experimental.pallas import tpu_sc as plsc

N_NODES = 100000
N_EDGES = 3200000
EDGE_IN = 4
HIDDEN = 128
NODE_OUT = 128
N_GRAPHS = 16
BN = 1000
GRID = N_NODES // BN

SC_NC = 2
SC_NS = 16
SC_NW = SC_NC * SC_NS
E_PER_W = N_EDGES // SC_NW
ROWW = 80
CHUNK = 2000
ROWS_PER_CHUNK = CHUNK // ROWW
NCHUNK = E_PER_W // CHUNK
NEG = -1e30
HALFN = N_NODES // 2
NGROUP = 8
E_PER_G = N_EDGES // SC_NC // NGROUP
CHUNK_M = 8000
NCHUNK_M = E_PER_G // CHUNK_M
VB = 4
NP_MAX = SC_NC * NGROUP


def _sc_body(col2d_h, payt_h, z_h,
             sump_h, maxp_h,
             idx2_v, pc0_v, pc1_v, pc2_v, pc3_v, pc4_v, idxf_v, val_v,
             table_v, t0_sh, t1_sh, t2_sh, t3_sh, t4_sh, sem):
    c = lax.axis_index("c")
    s = lax.axis_index("s")
    w = c * SC_NS + s
    pcs = [pc0_v, pc1_v, pc2_v, pc3_v, pc4_v]
    tbls = [t0_sh, t1_sh, t2_sh, t3_sh, t4_sh]

    @pl.when(s == 0)
    def _():
        for t in tbls:
            pltpu.sync_copy(z_h, t)
    plsc.subcore_barrier()

    def _sum_chunk(ci, carry):
        rowbase = (w * E_PER_W + ci * CHUNK) // ROWW
        ebase = w * E_PER_W + ci * CHUNK
        pltpu.sync_copy(col2d_h.at[pl.ds(rowbase, ROWS_PER_CHUNK), :], idx2_v)
        for q in range(5):
            pltpu.sync_copy(payt_h.at[q, pl.ds(ebase, CHUNK)], pcs[q])
        handles = []
        for j in range(ROWS_PER_CHUNK):
            for q in range(5):
                handles.append(pltpu.async_copy(
                    pcs[q].at[pl.ds(j * ROWW, ROWW)],
                    tbls[q].at[idx2_v.at[j]], sem, add=True))
        for h in handles:
            h.wait()
        return carry
    lax.fori_loop(0, NCHUNK, _sum_chunk, 0)
    plsc.subcore_barrier()

    def _sout(k, carry):
        gb = s + k * SC_NS
        @pl.when(gb < GRID)
        def _():
            for q in range(5):
                pltpu.sync_copy(tbls[q].at[pl.ds(gb * BN, BN)],
                                sump_h.at[gb, c, q, :])
        return carry
    lax.fori_loop(0, (GRID + SC_NS - 1) // SC_NS, _sout, 0)

    g = s // 2
    r = s % 2
    lo = r * HALFN
    neg16 = jnp.full((16,), NEG, jnp.float32)
    for f in range(EDGE_IN):
        def _init_tab(i, carry):
            table_v[pl.ds(i * 16, 16)] = neg16
            return carry
        lax.fori_loop(0, HALFN // 16, _init_tab, 0)

        def _max_chunk(ci, carry):
            ebase = c * (N_EDGES // SC_NC) + g * E_PER_G + ci * CHUNK_M
            pltpu.sync_copy(
                col2d_h.at[pl.ds(ebase // ROWW, CHUNK_M // ROWW), :], idxf_v)
            pltpu.sync_copy(payt_h.at[f, pl.ds(ebase, CHUNK_M)], val_v)

            def _vec(jb, vcarry):
                ils, vls, needs = [], [], []
                for t in range(VB):
                    jv = jb * VB + t
                    i16 = idxf_v[jv // (ROWW // 16),
                                 pl.ds((jv % (ROWW // 16)) * 16, 16)]
                    v16 = val_v[pl.ds(jv * 16, 16)]
                    il = i16 - lo
                    inr = il.astype(jnp.uint32) < jnp.uint32(HALFN)
                    old = plsc.load_gather(table_v, [il], mask=inr)
                    plsc.store_scatter(table_v, [il],
                                       jnp.maximum(old, v16), mask=inr)
                    ils.append(il)
                    vls.append(v16)
                    needs.append(inr)
                needs = tuple(
                    needs[t]
                    & (plsc.load_gather(table_v, [ils[t]], mask=needs[t])
                       < vls[t]) for t in range(VB))

                def _cond(ms):
                    acc = ms[0]
                    for t in range(1, VB):
                        acc = acc | ms[t]
                    return jnp.any(acc)

                def _body(ms):
                    out = []
                    for t in range(VB):
                        cur = plsc.load_gather(table_v, [ils[t]], mask=ms[t])
                        plsc.store_scatter(table_v, [ils[t]],
                                           jnp.maximum(cur, vls[t]),
                                           mask=ms[t])
                        out.append(
                            ms[t]
                            & (plsc.load_gather(table_v, [ils[t]],
                                                mask=ms[t]) < vls[t]))
                    return tuple(out)
                lax.while_loop(_cond, _body, needs)
                return vcarry
            lax.fori_loop(0, CHUNK_M // (16 * VB), _vec, 0)
            return carry
        lax.fori_loop(0, NCHUNK_M, _max_chunk, 0)

        p = c * NGROUP + g
        def _wout(k, carry):
            pltpu.sync_copy(table_v.at[pl.ds(k * BN, BN)],
                            maxp_h.at[r * (HALFN // BN) + k, p, f, :])
            return carry
        lax.fori_loop(0, HALFN // BN, _wout, 0)


def _sc_scatter(col2d, payt, z):
    mesh = plsc.VectorSubcoreMesh(core_axis_name="c", subcore_axis_name="s")
    return pl.kernel(
        _sc_body,
        out_type=[
            jax.ShapeDtypeStruct((GRID, SC_NC, 5, BN), jnp.float32),
            jax.ShapeDtypeStruct((GRID, NP_MAX, 4, BN), jnp.float32),
        ],
        mesh=mesh,
        compiler_params=pltpu.CompilerParams(use_tc_tiling_on_sc=False,
                                             needs_layout_passes=False),
        scratch_types=(
            [pltpu.VMEM((ROWS_PER_CHUNK, ROWW), jnp.int32)]
            + [pltpu.VMEM((CHUNK,), jnp.float32)] * 5
            + [pltpu.VMEM((CHUNK_M // ROWW, ROWW), jnp.int32),
               pltpu.VMEM((CHUNK_M,), jnp.float32),
               pltpu.VMEM((HALFN,), jnp.float32)]
            + [pltpu.VMEM_SHARED((N_NODES,), jnp.float32)] * 5
            + [pltpu.SemaphoreType.DMA]
        ),
    )(col2d, payt, z)


def _mlp_body(x_ref, sump_ref, maxp_ref, b2d_ref, u_ref,
              w1a_ref, w1b1_ref, w1b2_ref, w1b3_ref, w1c_ref, b1_ref,
              w2_ref, bias2_ref, out_ref):
    f32 = jnp.float32
    sall = jnp.sum(sump_ref[...], axis=1).reshape(5, BN)
    out1t = sall[:4]
    cntt = sall[4:5]
    mx = jnp.max(maxp_ref[...], axis=1).reshape(4, BN)
    out2t = jnp.where(cntt > 0.0, mx, 0.0)
    out3t = out1t * (1.0 / jnp.maximum(cntt, 1.0))

    tdot = lambda a, b: lax.dot_general(
        a, b, dimension_numbers=(((0,), (0,)), ((), ())),
        preferred_element_type=f32)
    acc = jnp.dot(x_ref[...], w1a_ref[...], preferred_element_type=f32)
    acc += b1_ref[...]
    acc += tdot(out1t, w1b1_ref[...])
    acc += tdot(out2t, w1b2_ref[...])
    acc += tdot(out3t, w1b3_ref[...])
    oh = (lax.broadcasted_iota(jnp.int32, (BN, N_GRAPHS), 1)
          == b2d_ref[...]).astype(f32)
    uw = jnp.dot(u_ref[...], w1c_ref[...], preferred_element_type=f32)
    acc += jnp.dot(oh, uw, preferred_element_type=f32)
    h1 = jnp.maximum(acc, 0.0)
    out_ref[...] = (jnp.dot(h1, w2_ref[...], preferred_element_type=f32)
                    + bias2_ref[...])


def _mlp_call(x, sump, maxp, b2d, u, W1, b1, W2, b2):
    p1 = sump.shape[1]
    p2 = maxp.shape[1]
    full = lambda shape: pl.BlockSpec(shape, lambda i: tuple(0 for _ in shape))
    return pl.pallas_call(
        _mlp_body,
        grid=(GRID,),
        in_specs=[
            pl.BlockSpec((BN, 128), lambda i: (i, 0)),
            pl.BlockSpec((1, p1, 5, BN), lambda i: (i, 0, 0, 0)),
            pl.BlockSpec((1, p2, 4, BN), lambda i: (i, 0, 0, 0)),
            pl.BlockSpec((BN, 1), lambda i: (i, 0)),
            full((N_GRAPHS, 16)),
            full((128, HIDDEN)),
            full((4, HIDDEN)),
            full((4, HIDDEN)),
            full((4, HIDDEN)),
            full((16, HIDDEN)),
            full((1, HIDDEN)),
            full((HIDDEN, NODE_OUT)),
            full((1, NODE_OUT)),
        ],
        out_specs=pl.BlockSpec((BN, NODE_OUT), lambda i: (i, 0)),
        out_shape=jax.ShapeDtypeStruct((N_NODES, NODE_OUT), jnp.float32),
        compiler_params=pltpu.CompilerParams(
            dimension_semantics=("arbitrary",)),
    )(x, sump, maxp, b2d, u, W1[:128], W1[128:132], W1[132:136],
      W1[136:140], W1[140:156], b1.reshape(1, HIDDEN), W2,
      b2.reshape(1, NODE_OUT))


def kernel(x, edge_index, edge_attr, u, batch, W1, b1, W2, b2):
    col2d = edge_index[1].astype(jnp.int32).reshape(N_EDGES // ROWW, ROWW)
    z = jnp.zeros((N_NODES,), jnp.float32)
    payt = jnp.concatenate(
        [edge_attr.T, jnp.ones((1, N_EDGES), jnp.float32)], axis=0)
    sump, maxp = _sc_scatter(col2d, payt, z)
    b2d = batch.astype(jnp.int32).reshape(N_NODES, 1)
    return _mlp_call(x, sump, maxp, b2d, u, W1, b1, W2, b2)
